# baseline (speedup 1.0000x reference)
import functools

import jax
import jax.numpy as jnp
from jax import lax
from jax.experimental import pallas as pl
from jax.experimental.pallas import tpu as pltpu
from jax.experimental.pallas import tpu_sc as plsc

M = 128


def _routing(group_indices, B, G, NB):
    gi = group_indices.astype(jnp.int32)
    oh = (gi[:, None] == jnp.arange(G, dtype=jnp.int32)[None, :]).astype(jnp.int32)
    counts = oh.sum(axis=0)
    rank = (jnp.cumsum(oh, axis=0) * oh).sum(axis=1) - 1
    nblk = -(-counts // M)
    blk_cum = jnp.cumsum(nblk)
    blk_start = blk_cum - nblk
    i = jnp.arange(NB, dtype=jnp.int32)
    bg = jnp.searchsorted(blk_cum, i, side="right").astype(jnp.int32)
    g_ids = jnp.arange(G, dtype=jnp.int32)
    g_last = jnp.max(jnp.where(counts > 0, g_ids, -1))
    bg = jnp.where(bg >= G, g_last, bg)
    first_tok = jnp.full((G,), 2**30, jnp.int32).at[gi].min(
        jnp.arange(B, dtype=jnp.int32), mode="drop"
    )
    fill = jnp.broadcast_to(first_tok[bg][:, None], (NB, M)).reshape(NB * M)
    slot = blk_start[gi] * M + rank
    row_idx = fill.at[slot].set(jnp.arange(B, dtype=jnp.int32))
    return row_idx, bg


def _sc_gather(x, row_idx, P, D):
    info = plsc.get_sparse_core_info()
    NC, NS = info.num_cores, info.num_subcores
    NW = NC * NS
    bpw = P // NW
    mesh = plsc.VectorSubcoreMesh(core_axis_name="c", subcore_axis_name="s")

    @functools.partial(
        pl.kernel,
        mesh=mesh,
        out_type=jax.ShapeDtypeStruct((P, D), jnp.float32),
        scratch_types=[
            pltpu.VMEM((bpw,), jnp.int32),
            pltpu.VMEM((bpw, D), jnp.float32),
            pltpu.SemaphoreType.DMA,
        ],
    )
    def k(x_hbm, idx_hbm, out_hbm, idx_v, rows_v, sem):
        wid = lax.axis_index("s") * NC + lax.axis_index("c")
        base = wid * bpw
        pltpu.sync_copy(idx_hbm.at[pl.ds(base, bpw)], idx_v)
        pltpu.async_copy(x_hbm.at[idx_v], rows_v, sem).wait()
        pltpu.sync_copy(rows_v, out_hbm.at[pl.ds(base, bpw)])

    return k(x, row_idx)


def _sc_scatter(y_sorted, row_idx, B, P, D):
    info = plsc.get_sparse_core_info()
    NC, NS = info.num_cores, info.num_subcores
    NW = NC * NS
    bpw = P // NW
    mesh = plsc.VectorSubcoreMesh(core_axis_name="c", subcore_axis_name="s")

    @functools.partial(
        pl.kernel,
        mesh=mesh,
        out_type=jax.ShapeDtypeStruct((B, D), jnp.float32),
        scratch_types=[
            pltpu.VMEM((bpw,), jnp.int32),
            pltpu.VMEM((bpw, D), jnp.float32),
            pltpu.SemaphoreType.DMA,
        ],
    )
    def k(ys_hbm, idx_hbm, out_hbm, idx_v, rows_v, sem):
        wid = lax.axis_index("s") * NC + lax.axis_index("c")
        base = wid * bpw
        pltpu.sync_copy(idx_hbm.at[pl.ds(base, bpw)], idx_v)
        pltpu.sync_copy(ys_hbm.at[pl.ds(base, bpw)], rows_v)
        pltpu.async_copy(rows_v, out_hbm.at[idx_v], sem).wait()

    return k(y_sorted, row_idx)


def _tc_grouped_matmul(x_sorted, W, b, block_group, NB, D):
    def body(bg_ref, x_ref, w_ref, b_ref, o_ref):
        o_ref[...] = (
            jnp.dot(x_ref[...], w_ref[0], preferred_element_type=jnp.float32)
            + b_ref[0]
        )

    G = W.shape[0]
    grid_spec = pltpu.PrefetchScalarGridSpec(
        num_scalar_prefetch=1,
        grid=(NB,),
        in_specs=[
            pl.BlockSpec((M, D), lambda i, bg: (i, 0)),
            pl.BlockSpec((1, D, D), lambda i, bg: (bg[i], 0, 0)),
            pl.BlockSpec((1, 1, D), lambda i, bg: (bg[i], 0, 0)),
        ],
        out_specs=pl.BlockSpec((M, D), lambda i, bg: (i, 0)),
    )
    return pl.pallas_call(
        body,
        grid_spec=grid_spec,
        out_shape=jax.ShapeDtypeStruct((NB * M, D), jnp.float32),
    )(block_group, x_sorted, W, b.reshape(G, 1, D))


def kernel(x, group_indices, W, b):
    B, D = x.shape
    G = W.shape[0]
    NB = B // M + G
    P = NB * M
    row_idx, block_group = _routing(group_indices, B, G, NB)
    x_sorted = _sc_gather(x, row_idx, P, D)
    y_sorted = _tc_grouped_matmul(x_sorted, W, b, block_group, NB, D)
    return _sc_scatter(y_sorted, row_idx, B, P, D)


# speedup vs baseline: 2.1115x; 2.1115x over previous
import functools

import jax
import jax.numpy as jnp
from jax import lax
from jax.experimental import pallas as pl
from jax.experimental.pallas import tpu as pltpu
from jax.experimental.pallas import tpu_sc as plsc

M = 128
L = 16


def _sc_route_dispatch(x, gi, B, D, G, NB, P):
    info = plsc.get_sparse_core_info()
    NC, NS = info.num_cores, info.num_subcores
    NW = NC * NS
    tpw = B // NW
    nv = tpw // L
    nv_all = B // L
    mesh = plsc.VectorSubcoreMesh(core_axis_name="c", subcore_axis_name="s")

    @functools.partial(
        pl.kernel,
        mesh=mesh,
        out_type=(
            jax.ShapeDtypeStruct((P, D), jnp.float32),
            jax.ShapeDtypeStruct((NB,), jnp.int32),
            jax.ShapeDtypeStruct((B,), jnp.int32),
        ),
        scratch_types=[
            pltpu.VMEM((B,), jnp.int32),
            pltpu.VMEM((tpw, D), jnp.float32),
            pltpu.VMEM((tpw,), jnp.int32),
            pltpu.VMEM((2 * L,), jnp.int32),
            pltpu.SemaphoreType.DMA,
            pltpu.SemaphoreType.DMA,
        ],
        compiler_params=pltpu.CompilerParams(needs_layout_passes=False),
    )
    def k(x_hbm, gi_hbm, xs_hbm, bg_hbm, slots_hbm, gi_v, rows_v, slot_v,
          bg_v, sem_x, sem_s):
        wid = lax.axis_index("s") * NC + lax.axis_index("c")
        base = wid * tpw
        cp_x = pltpu.async_copy(x_hbm.at[pl.ds(base, tpw)], rows_v, sem_x)
        pltpu.sync_copy(gi_hbm, gi_v)

        zero_s = jnp.int32(0)
        tots = [zero_s] * G
        prevs = [zero_s] * G

        def count_body(i, carry):
            tots, prevs = list(carry[0]), list(carry[1])
            vec = gi_v[pl.ds(i * L, L)]
            before = jnp.where(i < wid * nv, jnp.int32(1), jnp.int32(0))
            for g in range(G):
                pc = jnp.sum((vec == g).astype(jnp.int32))
                tots[g] = tots[g] + pc
                prevs[g] = prevs[g] + pc * before
            return (tuple(tots), tuple(prevs))

        tots, prevs = lax.fori_loop(0, nv_all, count_body,
                                    (tuple(tots), tuple(prevs)))

        nblk = [(tots[g] + (M - 1)) >> 7 for g in range(G)]
        blk_start = []
        acc = zero_s
        for g in range(G):
            blk_start.append(acc)
            acc = acc + nblk[g]
        blk_cum = [blk_start[g] + nblk[g] for g in range(G)]

        iota = lax.iota(jnp.int32, L)
        zero_v = jnp.zeros((L,), jnp.int32)
        for half in range(2):
            ivec = iota + half * L
            bg = zero_v
            for g in range(G):
                bg = bg + jnp.where(ivec >= blk_cum[g], 1, 0).astype(jnp.int32)
            bg_v[pl.ds(half * L, L)] = jnp.minimum(bg, G - 1)

        @pl.when(wid == 0)
        def _():
            pltpu.sync_copy(bg_v.at[pl.ds(0, NB)], bg_hbm)

        carry = [prevs[g] for g in range(G)]
        for v in range(nv):
            vec = gi_v[pl.ds((wid * nv + v) * L, L)]
            slot = zero_v
            for g in range(G):
                mi = (vec == g).astype(jnp.int32)
                rank = plsc.cumsum(mi) - mi + carry[g]
                slot = slot + mi * (blk_start[g] * M + rank)
                carry[g] = carry[g] + jnp.sum(mi)
            slot_v[pl.ds(v * L, L)] = slot

        pltpu.sync_copy(slot_v, slots_hbm.at[pl.ds(base, tpw)])
        cp_x.wait()
        pltpu.async_copy(rows_v, xs_hbm.at[slot_v], sem_s).wait()

    return k(x, gi)


def _sc_combine(y_sorted, slots, B, P, D):
    info = plsc.get_sparse_core_info()
    NC, NS = info.num_cores, info.num_subcores
    NW = NC * NS
    tpw = B // NW
    mesh = plsc.VectorSubcoreMesh(core_axis_name="c", subcore_axis_name="s")

    @functools.partial(
        pl.kernel,
        mesh=mesh,
        out_type=jax.ShapeDtypeStruct((B, D), jnp.float32),
        scratch_types=[
            pltpu.VMEM((tpw,), jnp.int32),
            pltpu.VMEM((tpw, D), jnp.float32),
            pltpu.SemaphoreType.DMA,
        ],
        compiler_params=pltpu.CompilerParams(needs_layout_passes=False),
    )
    def k(ys_hbm, slots_hbm, y_hbm, slot_v, rows_v, sem):
        wid = lax.axis_index("s") * NC + lax.axis_index("c")
        base = wid * tpw
        pltpu.sync_copy(slots_hbm.at[pl.ds(base, tpw)], slot_v)
        pltpu.async_copy(ys_hbm.at[slot_v], rows_v, sem).wait()
        pltpu.sync_copy(rows_v, y_hbm.at[pl.ds(base, tpw)])

    return k(y_sorted, slots)


def _tc_grouped_matmul(x_sorted, W, b, block_group, NB, D):
    def body(bg_ref, x_ref, w_ref, b_ref, o_ref):
        o_ref[...] = (
            jnp.dot(x_ref[...], w_ref[0], preferred_element_type=jnp.float32)
            + b_ref[0]
        )

    G = W.shape[0]
    grid_spec = pltpu.PrefetchScalarGridSpec(
        num_scalar_prefetch=1,
        grid=(NB,),
        in_specs=[
            pl.BlockSpec((M, D), lambda i, bg: (i, 0)),
            pl.BlockSpec((1, D, D), lambda i, bg: (bg[i], 0, 0)),
            pl.BlockSpec((1, 1, D), lambda i, bg: (bg[i], 0, 0)),
        ],
        out_specs=pl.BlockSpec((M, D), lambda i, bg: (i, 0)),
    )
    return pl.pallas_call(
        body,
        grid_spec=grid_spec,
        out_shape=jax.ShapeDtypeStruct((NB * M, D), jnp.float32),
    )(block_group, x_sorted, W, b.reshape(G, 1, D))


def kernel(x, group_indices, W, b):
    B, D = x.shape
    G = W.shape[0]
    NB = B // M + G
    P = NB * M
    gi = group_indices.astype(jnp.int32)
    x_sorted, block_group, slots = _sc_route_dispatch(x, gi, B, D, G, NB, P)
    y_sorted = _tc_grouped_matmul(x_sorted, W, b, block_group, NB, D)
    return _sc_combine(y_sorted, slots, B, P, D)
